# Initial kernel scaffold; baseline (speedup 1.0000x reference)
import jax
import jax.numpy as jnp
from jax.experimental import pallas as pl


def kernel(z, edge_index, W, att_src, att_dst, bias):
    raise NotImplementedError("write your pallas kernel here")


# speedup vs baseline: 1.0065x; 1.0065x over previous
import jax
import jax.numpy as jnp
from jax.experimental import pallas as pl

N = 10000
D = 256
L = 2


def _mm_kernel(x_ref, w_ref, o_ref):
    o_ref[...] = jax.lax.dot_general(
        x_ref[...], w_ref[...], (((1,), (1,)), ((), ())),
        preferred_element_type=jnp.float32)


def _matmul_wt(x, Wl):
    return pl.pallas_call(
        _mm_kernel,
        grid=(10,),
        in_specs=[pl.BlockSpec((N // 10, D), lambda i: (i, 0)),
                  pl.BlockSpec((D, D), lambda i: (0, 0))],
        out_specs=pl.BlockSpec((N // 10, D), lambda i: (i, 0)),
        out_shape=jax.ShapeDtypeStruct((N, D), jnp.float32),
    )(x, Wl)


def _gat_conv(x, s, d, Wl, a_s, a_d, bl):
    h = _matmul_wt(x, Wl)
    alpha_src_n = (h * a_s).sum(-1)
    alpha_dst_n = (h * a_d).sum(-1)
    e = jax.nn.leaky_relu(alpha_src_n[s] + alpha_dst_n[d], 0.2)
    emax = jax.ops.segment_max(e, d, num_segments=N)
    emax = jnp.where(jnp.isfinite(emax), emax, 0.0)
    ex = jnp.exp(e - emax[d])
    denom = jax.ops.segment_sum(ex, d, num_segments=N)
    alpha = ex / (denom[d] + 1e-16)
    out = jax.ops.segment_sum(h[s] * alpha[:, None], d, num_segments=N)
    return out + bl


def kernel(z, edge_index, W, att_src, att_dst, bias):
    loop = jnp.arange(N, dtype=edge_index.dtype)
    s = jnp.concatenate([edge_index[0], loop])
    d = jnp.concatenate([edge_index[1], loop])
    out = z
    for l in range(L):
        out = jax.nn.relu(_gat_conv(out, s, d, W[l], att_src[l], att_dst[l], bias[l]))
    out = _gat_conv(out, s, d, W[L - 1], att_src[L - 1], att_dst[L - 1], bias[L - 1])
    return out
